# Initial kernel scaffold; baseline (speedup 1.0000x reference)
import jax
import jax.numpy as jnp
from jax.experimental import pallas as pl


def kernel(rgb_fea, ir_fea, edge_index, pos_emb, Ws, Bs, ln_w, ln_b):
    raise NotImplementedError("write your pallas kernel here")


# speedup vs baseline: 190.3287x; 190.3287x over previous
import functools
import numpy as np
import jax
import jax.numpy as jnp
from jax import lax
from jax.experimental import pallas as pl
from jax.experimental.pallas import tpu as pltpu
from jax.experimental.pallas import tpu_sc as plsc

DM = 256
NL = 8
NN = 2048
NE = 65536
NT = NE + NN
BSZ = 4
EPT = NT // 16
ROWS = EPT // 128
QROWS = 512
QN = QROWS * NN
AFLAT = NN * NN


def _sc_build_at(sc_src, sc_dst):
    mesh = plsc.VectorSubcoreMesh(core_axis_name="c", subcore_axis_name="s")

    def body(src_hbm, dst_hbm, at_hbm, src_v, dst_v, idx_v, val_v, dis_v,
             zero_v, qbuf, deg_s):
        c = lax.axis_index("c")
        s = lax.axis_index("s")

        def fill_zero(i, carry):
            zero_v[pl.ds(i * 16, 16)] = jnp.zeros((16,), jnp.float32)
            return carry
        lax.fori_loop(0, 512, fill_zero, 0)

        def fill_ones(r, carry):
            for k in range(8):
                val_v[r, pl.ds(k * 16, 16)] = jnp.full((16,), 1.0, jnp.float32)
            return carry
        lax.fori_loop(0, ROWS, fill_ones, 0)

        pltpu.sync_copy(src_hbm.at[pl.ds(s * EPT, EPT)], src_v)
        pltpu.sync_copy(dst_hbm.at[pl.ds(s * EPT, EPT)], dst_v)

        pltpu.sync_copy(zero_v.at[pl.ds(0, 128)], deg_s.at[pl.ds(s * 128, 128)])

        def dst_rows(r, carry):
            for k in range(8):
                idx_v[r, pl.ds(k * 16, 16)] = dst_v[pl.ds(r * 128 + k * 16, 16)]
            return carry
        lax.fori_loop(0, ROWS, dst_rows, 0)
        plsc.subcore_barrier()

        def deg_scatter(j, carry):
            pltpu.sync_copy(val_v.at[j], deg_s.at[idx_v.at[j]], add=True)
            return carry
        lax.fori_loop(0, ROWS, deg_scatter, 0)
        plsc.subcore_barrier()

        pltpu.sync_copy(deg_s, dis_v)

        def newton(i, carry):
            d = dis_v[pl.ds(i * 16, 16)]
            xi = lax.bitcast_convert_type(d, jnp.int32)
            yi = jnp.int32(0x5F3759DF) - lax.shift_right_logical(xi, 1)
            y = lax.bitcast_convert_type(yi, jnp.float32)
            for _ in range(3):
                y = y * (1.5 - 0.5 * d * y * y)
            dis_v[pl.ds(i * 16, 16)] = y
            return carry
        lax.fori_loop(0, NN // 16, newton, 0)

        for q in range(2):
            panel = 2 * c + q
            qlo = panel * QROWS

            for k in range(8):
                pltpu.sync_copy(
                    zero_v, qbuf.at[pl.ds(s * (QN // 16) + k * 8192, 8192)])
            plsc.subcore_barrier()

            def compute(r, carry):
                for k in range(8):
                    sl = pl.ds(k * 16, 16)
                    fl = pl.ds(r * 128 + k * 16, 16)
                    sv = src_v[fl]
                    dv = dst_v[fl]
                    nsv = plsc.load_gather(dis_v, [sv])
                    ndv = plsc.load_gather(dis_v, [dv])
                    nrm = nsv * ndv
                    inq = (sv >= qlo) & (sv < qlo + QROWS)
                    idx_v[r, sl] = jnp.where(inq, (sv - qlo) * NN + dv, QN + dv)
                    val_v[r, sl] = jnp.where(inq, nrm, 0.0)
                return carry
            lax.fori_loop(0, ROWS, compute, 0)

            def scatter(j, carry):
                pltpu.sync_copy(val_v.at[j], qbuf.at[idx_v.at[j]], add=True)
                return carry
            lax.fori_loop(0, ROWS, scatter, 0)
            plsc.subcore_barrier()

            pltpu.sync_copy(
                qbuf.at[pl.ds(s * (QN // 16), QN // 16)],
                at_hbm.at[pl.ds(panel * QN + s * (QN // 16), QN // 16)])

    return pl.kernel(
        body,
        out_type=jax.ShapeDtypeStruct((AFLAT,), jnp.float32),
        mesh=mesh,
        compiler_params=pltpu.CompilerParams(needs_layout_passes=False),
        scratch_types=[
            pltpu.VMEM((EPT,), jnp.int32),
            pltpu.VMEM((EPT,), jnp.int32),
            pltpu.VMEM((ROWS, 128), jnp.int32),
            pltpu.VMEM((ROWS, 128), jnp.float32),
            pltpu.VMEM((NN,), jnp.float32),
            pltpu.VMEM((8192,), jnp.float32),
            pltpu.VMEM_SHARED((QN + NN,), jnp.float32),
            pltpu.VMEM_SHARED((NN,), jnp.float32),
        ],
    )(sc_src, sc_dst)


def _gcn_body(xt_ref, pos_ref, wt_ref, bs_ref, at_ref, lnw_ref, lnb_ref,
              out_ref):
    x = 0.25 * (xt_ref[0, 0] + xt_ref[0, 1] + xt_ref[0, 2] + xt_ref[0, 3])
    x = x + pos_ref[...]
    a = at_ref[...]
    for i in range(NL):
        z = jnp.dot(wt_ref[i], x, preferred_element_type=jnp.float32)
        g = jnp.dot(z, a, preferred_element_type=jnp.float32)
        x = jnp.maximum(g + bs_ref[i], 0.0)
    mu = jnp.mean(x, axis=0, keepdims=True)
    xc = x - mu
    var = jnp.mean(xc * xc, axis=0, keepdims=True)
    xn = xc * lax.rsqrt(var + 1e-5)
    out_ref[0] = xn * lnw_ref[...] + lnb_ref[...]


def _resize_body(y_ref, g_ref, out_ref):
    out_ref[0, 0] = jnp.dot(y_ref[0], g_ref[...],
                            preferred_element_type=jnp.float32)


def _resize_matrix():
    sf = (np.arange(64) + 0.5) / 2.0 - 0.5
    w = np.maximum(0.0, 1.0 - np.abs(sf[None, :] - np.arange(32)[:, None]))
    w = w / w.sum(axis=0, keepdims=True)
    return w.T.astype(np.float32)


_R = _resize_matrix()
_G = np.kron(_R, _R).T.copy()


def kernel(rgb_fea, ir_fea, edge_index, pos_emb, Ws, Bs, ln_w, ln_b):
    def prep(t):
        return t.reshape(BSZ, DM, 32, 2, 32, 2).transpose(0, 3, 5, 1, 2, 4) \
                .reshape(BSZ, 4, DM, 1024)
    xt = jnp.concatenate([prep(rgb_fea), prep(ir_fea)], axis=-1)
    pos_t = jnp.transpose(pos_emb[0])
    wt = jnp.transpose(Ws, (0, 2, 1))
    bs2 = Bs[:, :, None]
    lnw2 = ln_w[:, None]
    lnb2 = ln_b[:, None]

    loop = jnp.arange(NN, dtype=jnp.int32)
    sc_src = jnp.concatenate([edge_index[0], loop])
    sc_dst = jnp.concatenate([edge_index[1], loop])

    at = _sc_build_at(sc_src, sc_dst).reshape(NN, NN)

    y = pl.pallas_call(
        _gcn_body,
        grid=(BSZ,),
        in_specs=[
            pl.BlockSpec((1, 4, DM, NN), lambda b: (b, 0, 0, 0)),
            pl.BlockSpec((DM, NN), lambda b: (0, 0)),
            pl.BlockSpec((NL, DM, DM), lambda b: (0, 0, 0)),
            pl.BlockSpec((NL, DM, 1), lambda b: (0, 0, 0)),
            pl.BlockSpec((NN, NN), lambda b: (0, 0)),
            pl.BlockSpec((DM, 1), lambda b: (0, 0)),
            pl.BlockSpec((DM, 1), lambda b: (0, 0)),
        ],
        out_specs=pl.BlockSpec((1, DM, NN), lambda b: (b, 0, 0)),
        out_shape=jax.ShapeDtypeStruct((BSZ, DM, NN), jnp.float32),
    )(xt, pos_t, wt, bs2, at, lnw2, lnb2)

    out = pl.pallas_call(
        _resize_body,
        grid=(BSZ, 2),
        in_specs=[
            pl.BlockSpec((1, DM, 1024), lambda b, m: (b, 0, m)),
            pl.BlockSpec((1024, 4096), lambda b, m: (0, 0)),
        ],
        out_specs=pl.BlockSpec((1, 1, DM, 4096), lambda b, m: (b, m, 0, 0)),
        out_shape=jax.ShapeDtypeStruct((BSZ, 2, DM, 4096), jnp.float32),
    )(y, jnp.asarray(_G))

    rgb_out = out[:, 0].reshape(BSZ, DM, 64, 64)
    ir_out = out[:, 1].reshape(BSZ, DM, 64, 64)
    return rgb_out, ir_out


# speedup vs baseline: 269.4533x; 1.4157x over previous
import functools
import numpy as np
import jax
import jax.numpy as jnp
from jax import lax
from jax.experimental import pallas as pl
from jax.experimental.pallas import tpu as pltpu
from jax.experimental.pallas import tpu_sc as plsc

DM = 256
NL = 8
NN = 2048
NE = 65536
NT = NE + NN
BSZ = 4
EPT = NT // 16
ROWS = EPT // 128
QROWS = 512
QN = QROWS * NN
AFLAT = NN * NN


def _sc_build_at(sc_src, sc_dst):
    mesh = plsc.VectorSubcoreMesh(core_axis_name="c", subcore_axis_name="s")

    def body(src_hbm, dst_hbm, at_hbm, src_v, dst_v, idx_v, val_v, dis_v,
             zero_v, qbuf, deg_s):
        c = lax.axis_index("c")
        s = lax.axis_index("s")

        def fill_zero(i, carry):
            zero_v[pl.ds(i * 16, 16)] = jnp.zeros((16,), jnp.float32)
            return carry
        lax.fori_loop(0, 512, fill_zero, 0)

        def fill_ones(r, carry):
            for k in range(8):
                val_v[r, pl.ds(k * 16, 16)] = jnp.full((16,), 1.0, jnp.float32)
            return carry
        lax.fori_loop(0, ROWS, fill_ones, 0)

        pltpu.sync_copy(src_hbm.at[pl.ds(s * EPT, EPT)], src_v)
        pltpu.sync_copy(dst_hbm.at[pl.ds(s * EPT, EPT)], dst_v)

        pltpu.sync_copy(zero_v.at[pl.ds(0, 128)], deg_s.at[pl.ds(s * 128, 128)])

        def dst_rows(r, carry):
            for k in range(8):
                idx_v[r, pl.ds(k * 16, 16)] = dst_v[pl.ds(r * 128 + k * 16, 16)]
            return carry
        lax.fori_loop(0, ROWS, dst_rows, 0)
        plsc.subcore_barrier()

        def deg_scatter(j, carry):
            pltpu.sync_copy(val_v.at[j], deg_s.at[idx_v.at[j]], add=True)
            return carry
        lax.fori_loop(0, ROWS, deg_scatter, 0)
        plsc.subcore_barrier()

        pltpu.sync_copy(deg_s, dis_v)

        def newton(i, carry):
            d = dis_v[pl.ds(i * 16, 16)]
            xi = lax.bitcast_convert_type(d, jnp.int32)
            yi = jnp.int32(0x5F3759DF) - lax.shift_right_logical(xi, 1)
            y = lax.bitcast_convert_type(yi, jnp.float32)
            for _ in range(3):
                y = y * (1.5 - 0.5 * d * y * y)
            dis_v[pl.ds(i * 16, 16)] = y
            return carry
        lax.fori_loop(0, NN // 16, newton, 0)

        for q in range(2):
            panel = 2 * c + q
            qlo = panel * QROWS

            for k in range(8):
                pltpu.sync_copy(
                    zero_v, qbuf.at[pl.ds(s * (QN // 16) + k * 8192, 8192)])
            plsc.subcore_barrier()

            def compute(r, carry):
                for k in range(8):
                    sl = pl.ds(k * 16, 16)
                    fl = pl.ds(r * 128 + k * 16, 16)
                    sv = src_v[fl]
                    dv = dst_v[fl]
                    nsv = plsc.load_gather(dis_v, [sv])
                    ndv = plsc.load_gather(dis_v, [dv])
                    nrm = nsv * ndv
                    inq = (sv >= qlo) & (sv < qlo + QROWS)
                    idx_v[r, sl] = jnp.where(inq, (sv - qlo) * NN + dv, QN + dv)
                    val_v[r, sl] = jnp.where(inq, nrm, 0.0)
                return carry
            lax.fori_loop(0, ROWS, compute, 0)

            def scatter(j, carry):
                pltpu.sync_copy(val_v.at[j], qbuf.at[idx_v.at[j]], add=True)
                return carry
            lax.fori_loop(0, ROWS, scatter, 0)
            plsc.subcore_barrier()

            pltpu.sync_copy(
                qbuf.at[pl.ds(s * (QN // 16), QN // 16)],
                at_hbm.at[pl.ds(panel * QN + s * (QN // 16), QN // 16)])

    return pl.kernel(
        body,
        out_type=jax.ShapeDtypeStruct((AFLAT,), jnp.float32),
        mesh=mesh,
        compiler_params=pltpu.CompilerParams(needs_layout_passes=False),
        scratch_types=[
            pltpu.VMEM((EPT,), jnp.int32),
            pltpu.VMEM((EPT,), jnp.int32),
            pltpu.VMEM((ROWS, 128), jnp.int32),
            pltpu.VMEM((ROWS, 128), jnp.float32),
            pltpu.VMEM((NN,), jnp.float32),
            pltpu.VMEM((8192,), jnp.float32),
            pltpu.VMEM_SHARED((QN + NN,), jnp.float32),
            pltpu.VMEM_SHARED((NN,), jnp.float32),
        ],
    )(sc_src, sc_dst)


_PH = np.zeros((64, 32), np.float32)
_PH[np.arange(64), np.arange(64) // 2] = 0.5
_GP = np.kron(_PH, _PH)


def _pool_body(rgb_ref, ir_ref, pos_ref, gp_ref, out_ref):
    gp = gp_ref[...]
    x = jnp.concatenate(
        [jnp.dot(rgb_ref[0], gp, preferred_element_type=jnp.float32),
         jnp.dot(ir_ref[0], gp, preferred_element_type=jnp.float32)], axis=1)
    out_ref[0] = x + pos_ref[...]


def _gcn_body(x_ref, wt_ref, bs_ref, at_ref, lnw_ref, lnb_ref, out_ref):
    x = x_ref[0]
    a = at_ref[...]
    for i in range(NL):
        z = jnp.dot(wt_ref[i], x, preferred_element_type=jnp.float32)
        g = jnp.dot(z, a, preferred_element_type=jnp.float32)
        x = jnp.maximum(g + bs_ref[i], 0.0)
    mu = jnp.mean(x, axis=0, keepdims=True)
    xc = x - mu
    var = jnp.mean(xc * xc, axis=0, keepdims=True)
    xn = xc * lax.rsqrt(var + 1e-5)
    out_ref[0] = xn * lnw_ref[...] + lnb_ref[...]


def _resize_body(y_ref, g_ref, rgb_ref, ir_ref):
    rgb_ref[0] = jnp.dot(y_ref[0, :, :1024], g_ref[...],
                         preferred_element_type=jnp.float32)
    ir_ref[0] = jnp.dot(y_ref[0, :, 1024:], g_ref[...],
                        preferred_element_type=jnp.float32)


def _resize_matrix():
    sf = (np.arange(64) + 0.5) / 2.0 - 0.5
    w = np.maximum(0.0, 1.0 - np.abs(sf[None, :] - np.arange(32)[:, None]))
    w = w / w.sum(axis=0, keepdims=True)
    return w.T.astype(np.float32)


_R = _resize_matrix()
_G = np.kron(_R, _R).T.copy()


def kernel(rgb_fea, ir_fea, edge_index, pos_emb, Ws, Bs, ln_w, ln_b):
    pos_t = jnp.transpose(pos_emb[0])
    wt = jnp.transpose(Ws, (0, 2, 1))
    bs2 = Bs[:, :, None]
    lnw2 = ln_w[:, None]
    lnb2 = ln_b[:, None]

    loop = jnp.arange(NN, dtype=jnp.int32)
    sc_src = jnp.concatenate([edge_index[0], loop])
    sc_dst = jnp.concatenate([edge_index[1], loop])

    at = _sc_build_at(sc_src, sc_dst).reshape(NN, NN)

    x0 = pl.pallas_call(
        _pool_body,
        grid=(BSZ,),
        in_specs=[
            pl.BlockSpec((1, DM, 4096), lambda b: (b, 0, 0)),
            pl.BlockSpec((1, DM, 4096), lambda b: (b, 0, 0)),
            pl.BlockSpec((DM, NN), lambda b: (0, 0)),
            pl.BlockSpec((4096, 1024), lambda b: (0, 0)),
        ],
        out_specs=pl.BlockSpec((1, DM, NN), lambda b: (b, 0, 0)),
        out_shape=jax.ShapeDtypeStruct((BSZ, DM, NN), jnp.float32),
    )(rgb_fea.reshape(BSZ, DM, 4096), ir_fea.reshape(BSZ, DM, 4096),
      pos_t, jnp.asarray(_GP))

    y = pl.pallas_call(
        _gcn_body,
        grid=(BSZ,),
        in_specs=[
            pl.BlockSpec((1, DM, NN), lambda b: (b, 0, 0)),
            pl.BlockSpec((NL, DM, DM), lambda b: (0, 0, 0)),
            pl.BlockSpec((NL, DM, 1), lambda b: (0, 0, 0)),
            pl.BlockSpec((NN, NN), lambda b: (0, 0)),
            pl.BlockSpec((DM, 1), lambda b: (0, 0)),
            pl.BlockSpec((DM, 1), lambda b: (0, 0)),
        ],
        out_specs=pl.BlockSpec((1, DM, NN), lambda b: (b, 0, 0)),
        out_shape=jax.ShapeDtypeStruct((BSZ, DM, NN), jnp.float32),
    )(x0, wt, bs2, at, lnw2, lnb2)

    rgb_out, ir_out = pl.pallas_call(
        _resize_body,
        grid=(BSZ,),
        in_specs=[
            pl.BlockSpec((1, DM, NN), lambda b: (b, 0, 0)),
            pl.BlockSpec((1024, 4096), lambda b: (0, 0)),
        ],
        out_specs=[
            pl.BlockSpec((1, DM, 4096), lambda b: (b, 0, 0)),
            pl.BlockSpec((1, DM, 4096), lambda b: (b, 0, 0)),
        ],
        out_shape=[
            jax.ShapeDtypeStruct((BSZ, DM, 4096), jnp.float32),
            jax.ShapeDtypeStruct((BSZ, DM, 4096), jnp.float32),
        ],
    )(y, jnp.asarray(_G))

    return (rgb_out.reshape(BSZ, DM, 64, 64), ir_out.reshape(BSZ, DM, 64, 64))


# speedup vs baseline: 269.6928x; 1.0009x over previous
import functools
import numpy as np
import jax
import jax.numpy as jnp
from jax import lax
from jax.experimental import pallas as pl
from jax.experimental.pallas import tpu as pltpu
from jax.experimental.pallas import tpu_sc as plsc

DM = 256
NL = 8
NN = 2048
NE = 65536
NT = NE + NN
BSZ = 4
EPT = NT // 16
ROWS = EPT // 128
QROWS = 512
QN = QROWS * NN
AFLAT = NN * NN


def _sc_build_at(sc_src, sc_dst):
    mesh = plsc.VectorSubcoreMesh(core_axis_name="c", subcore_axis_name="s")

    def body(src_hbm, dst_hbm, at_hbm, src_v, dst_v, idx_v, val_v, dis_v,
             zero_v, qbuf, deg_s, sem):
        c = lax.axis_index("c")
        s = lax.axis_index("s")

        def fill_zero(i, carry):
            zero_v[pl.ds(i * 16, 16)] = jnp.zeros((16,), jnp.float32)
            return carry
        lax.fori_loop(0, 512, fill_zero, 0)

        def fill_ones(r, carry):
            for k in range(8):
                val_v[r, pl.ds(k * 16, 16)] = jnp.full((16,), 1.0, jnp.float32)
            return carry
        lax.fori_loop(0, ROWS, fill_ones, 0)

        pltpu.sync_copy(src_hbm.at[pl.ds(s * EPT, EPT)], src_v)
        pltpu.sync_copy(dst_hbm.at[pl.ds(s * EPT, EPT)], dst_v)

        pltpu.sync_copy(zero_v.at[pl.ds(0, 128)], deg_s.at[pl.ds(s * 128, 128)])

        def dst_rows(r, carry):
            for k in range(8):
                idx_v[r, pl.ds(k * 16, 16)] = dst_v[pl.ds(r * 128 + k * 16, 16)]
            return carry
        lax.fori_loop(0, ROWS, dst_rows, 0)
        plsc.subcore_barrier()

        degs = [pltpu.async_copy(val_v.at[j], deg_s.at[idx_v.at[j]], sem,
                                 add=True) for j in range(ROWS)]
        for d in degs:
            d.wait()
        plsc.subcore_barrier()

        pltpu.sync_copy(deg_s, dis_v)

        def newton(i, carry):
            d = dis_v[pl.ds(i * 16, 16)]
            xi = lax.bitcast_convert_type(d, jnp.int32)
            yi = jnp.int32(0x5F3759DF) - lax.shift_right_logical(xi, 1)
            y = lax.bitcast_convert_type(yi, jnp.float32)
            for _ in range(3):
                y = y * (1.5 - 0.5 * d * y * y)
            dis_v[pl.ds(i * 16, 16)] = y
            return carry
        lax.fori_loop(0, NN // 16, newton, 0)

        def compute_panel(qlo):
            def compute(r, carry):
                for k in range(8):
                    sl = pl.ds(k * 16, 16)
                    fl = pl.ds(r * 128 + k * 16, 16)
                    sv = src_v[fl]
                    dv = dst_v[fl]
                    nsv = plsc.load_gather(dis_v, [sv])
                    ndv = plsc.load_gather(dis_v, [dv])
                    nrm = nsv * ndv
                    inq = (sv >= qlo) & (sv < qlo + QROWS)
                    idx_v[r, sl] = jnp.where(inq, (sv - qlo) * NN + dv, QN + dv)
                    val_v[r, sl] = jnp.where(inq, nrm, 0.0)
                return carry
            lax.fori_loop(0, ROWS, compute, 0)

        wout = None
        for q in range(2):
            panel = 2 * c + q
            qlo = panel * QROWS

            compute_panel(qlo)
            if wout is not None:
                wout.wait()

            zs = [pltpu.async_copy(
                      zero_v, qbuf.at[pl.ds(s * (QN // 16) + k * 8192, 8192)],
                      sem) for k in range(8)]
            for z in zs:
                z.wait()
            plsc.subcore_barrier()

            scs = [pltpu.async_copy(val_v.at[j], qbuf.at[idx_v.at[j]], sem,
                                    add=True) for j in range(ROWS)]
            for d in scs:
                d.wait()
            plsc.subcore_barrier()

            wout = pltpu.async_copy(
                qbuf.at[pl.ds(s * (QN // 16), QN // 16)],
                at_hbm.at[pl.ds(panel * QN + s * (QN // 16), QN // 16)], sem)
        wout.wait()

    return pl.kernel(
        body,
        out_type=jax.ShapeDtypeStruct((AFLAT,), jnp.float32),
        mesh=mesh,
        compiler_params=pltpu.CompilerParams(needs_layout_passes=False),
        scratch_types=[
            pltpu.VMEM((EPT,), jnp.int32),
            pltpu.VMEM((EPT,), jnp.int32),
            pltpu.VMEM((ROWS, 128), jnp.int32),
            pltpu.VMEM((ROWS, 128), jnp.float32),
            pltpu.VMEM((NN,), jnp.float32),
            pltpu.VMEM((8192,), jnp.float32),
            pltpu.VMEM_SHARED((QN + NN,), jnp.float32),
            pltpu.VMEM_SHARED((NN,), jnp.float32),
            pltpu.SemaphoreType.DMA,
        ],
    )(sc_src, sc_dst)


_PH = np.zeros((64, 32), np.float32)
_PH[np.arange(64), np.arange(64) // 2] = 0.5
_GP = np.kron(_PH, _PH)


def _pool_body(rgb_ref, ir_ref, pos_ref, gp_ref, out_ref):
    gp = gp_ref[...]
    x = jnp.concatenate(
        [jnp.dot(rgb_ref[0], gp, preferred_element_type=jnp.float32),
         jnp.dot(ir_ref[0], gp, preferred_element_type=jnp.float32)], axis=1)
    out_ref[0] = x + pos_ref[...]


def _gcn_body(x_ref, wt_ref, bs_ref, at_ref, lnw_ref, lnb_ref, out_ref):
    x = x_ref[0]
    a = at_ref[...]
    for i in range(NL):
        z = jnp.dot(wt_ref[i], x, preferred_element_type=jnp.float32)
        g = jnp.dot(z, a, preferred_element_type=jnp.float32)
        x = jnp.maximum(g + bs_ref[i], 0.0)
    mu = jnp.mean(x, axis=0, keepdims=True)
    xc = x - mu
    var = jnp.mean(xc * xc, axis=0, keepdims=True)
    xn = xc * lax.rsqrt(var + 1e-5)
    out_ref[0] = xn * lnw_ref[...] + lnb_ref[...]


def _resize_body(y_ref, g_ref, rgb_ref, ir_ref):
    rgb_ref[0] = jnp.dot(y_ref[0, :, :1024], g_ref[...],
                         preferred_element_type=jnp.float32)
    ir_ref[0] = jnp.dot(y_ref[0, :, 1024:], g_ref[...],
                        preferred_element_type=jnp.float32)


def _resize_matrix():
    sf = (np.arange(64) + 0.5) / 2.0 - 0.5
    w = np.maximum(0.0, 1.0 - np.abs(sf[None, :] - np.arange(32)[:, None]))
    w = w / w.sum(axis=0, keepdims=True)
    return w.T.astype(np.float32)


_R = _resize_matrix()
_G = np.kron(_R, _R).T.copy()


def kernel(rgb_fea, ir_fea, edge_index, pos_emb, Ws, Bs, ln_w, ln_b):
    pos_t = jnp.transpose(pos_emb[0])
    wt = jnp.transpose(Ws, (0, 2, 1))
    bs2 = Bs[:, :, None]
    lnw2 = ln_w[:, None]
    lnb2 = ln_b[:, None]

    loop = jnp.arange(NN, dtype=jnp.int32)
    sc_src = jnp.concatenate([edge_index[0], loop])
    sc_dst = jnp.concatenate([edge_index[1], loop])

    at = _sc_build_at(sc_src, sc_dst).reshape(NN, NN)

    x0 = pl.pallas_call(
        _pool_body,
        grid=(BSZ,),
        in_specs=[
            pl.BlockSpec((1, DM, 4096), lambda b: (b, 0, 0)),
            pl.BlockSpec((1, DM, 4096), lambda b: (b, 0, 0)),
            pl.BlockSpec((DM, NN), lambda b: (0, 0)),
            pl.BlockSpec((4096, 1024), lambda b: (0, 0)),
        ],
        out_specs=pl.BlockSpec((1, DM, NN), lambda b: (b, 0, 0)),
        out_shape=jax.ShapeDtypeStruct((BSZ, DM, NN), jnp.float32),
    )(rgb_fea.reshape(BSZ, DM, 4096), ir_fea.reshape(BSZ, DM, 4096),
      pos_t, jnp.asarray(_GP))

    y = pl.pallas_call(
        _gcn_body,
        grid=(BSZ,),
        in_specs=[
            pl.BlockSpec((1, DM, NN), lambda b: (b, 0, 0)),
            pl.BlockSpec((NL, DM, DM), lambda b: (0, 0, 0)),
            pl.BlockSpec((NL, DM, 1), lambda b: (0, 0, 0)),
            pl.BlockSpec((NN, NN), lambda b: (0, 0)),
            pl.BlockSpec((DM, 1), lambda b: (0, 0)),
            pl.BlockSpec((DM, 1), lambda b: (0, 0)),
        ],
        out_specs=pl.BlockSpec((1, DM, NN), lambda b: (b, 0, 0)),
        out_shape=jax.ShapeDtypeStruct((BSZ, DM, NN), jnp.float32),
    )(x0, wt, bs2, at, lnw2, lnb2)

    rgb_out, ir_out = pl.pallas_call(
        _resize_body,
        grid=(BSZ,),
        in_specs=[
            pl.BlockSpec((1, DM, NN), lambda b: (b, 0, 0)),
            pl.BlockSpec((1024, 4096), lambda b: (0, 0)),
        ],
        out_specs=[
            pl.BlockSpec((1, DM, 4096), lambda b: (b, 0, 0)),
            pl.BlockSpec((1, DM, 4096), lambda b: (b, 0, 0)),
        ],
        out_shape=[
            jax.ShapeDtypeStruct((BSZ, DM, 4096), jnp.float32),
            jax.ShapeDtypeStruct((BSZ, DM, 4096), jnp.float32),
        ],
    )(y, jnp.asarray(_G))

    return (rgb_out.reshape(BSZ, DM, 64, 64), ir_out.reshape(BSZ, DM, 64, 64))


# speedup vs baseline: 288.1671x; 1.0685x over previous
import functools
import numpy as np
import jax
import jax.numpy as jnp
from jax import lax
from jax.experimental import pallas as pl
from jax.experimental.pallas import tpu as pltpu
from jax.experimental.pallas import tpu_sc as plsc

DM = 256
NL = 8
NN = 2048
NE = 65536
NT = NE + NN
BSZ = 4
EPT = NT // 16
ROWS = EPT // 128
QROWS = 512
QN = QROWS * NN
AFLAT = NN * NN


def _sc_build_at(sc_src, sc_dst):
    mesh = plsc.VectorSubcoreMesh(core_axis_name="c", subcore_axis_name="s")

    def body(src_hbm, dst_hbm, at_hbm, src_v, dst_v, idx_v, val_v, dis_v,
             zero_v, bounce_a, bounce_b, qbuf, deg_s, sem, sem_w):
        c = lax.axis_index("c")
        s = lax.axis_index("s")

        def fill_zero(i, carry):
            zero_v[pl.ds(i * 16, 16)] = jnp.zeros((16,), jnp.float32)
            return carry
        lax.fori_loop(0, 512, fill_zero, 0)

        def fill_ones(r, carry):
            for k in range(8):
                val_v[r, pl.ds(k * 16, 16)] = jnp.full((16,), 1.0, jnp.float32)
            return carry
        lax.fori_loop(0, ROWS, fill_ones, 0)

        pltpu.sync_copy(src_hbm.at[pl.ds(s * EPT, EPT)], src_v)
        pltpu.sync_copy(dst_hbm.at[pl.ds(s * EPT, EPT)], dst_v)

        pltpu.sync_copy(zero_v.at[pl.ds(0, 128)], deg_s.at[pl.ds(s * 128, 128)])

        def dst_rows(r, carry):
            for k in range(8):
                idx_v[r, pl.ds(k * 16, 16)] = dst_v[pl.ds(r * 128 + k * 16, 16)]
            return carry
        lax.fori_loop(0, ROWS, dst_rows, 0)
        plsc.subcore_barrier()

        degs = [pltpu.async_copy(val_v.at[j], deg_s.at[idx_v.at[j]], sem,
                                 add=True) for j in range(ROWS)]
        for d in degs:
            d.wait()
        plsc.subcore_barrier()

        pltpu.sync_copy(deg_s, dis_v)

        def newton(i, carry):
            d = dis_v[pl.ds(i * 16, 16)]
            xi = lax.bitcast_convert_type(d, jnp.int32)
            yi = jnp.int32(0x5F3759DF) - lax.shift_right_logical(xi, 1)
            y = lax.bitcast_convert_type(yi, jnp.float32)
            for _ in range(3):
                y = y * (1.5 - 0.5 * d * y * y)
            dis_v[pl.ds(i * 16, 16)] = y
            return carry
        lax.fori_loop(0, NN // 16, newton, 0)

        def compute_panel(qlo):
            def compute(r, carry):
                for k in range(8):
                    sl = pl.ds(k * 16, 16)
                    fl = pl.ds(r * 128 + k * 16, 16)
                    sv = src_v[fl]
                    dv = dst_v[fl]
                    nsv = plsc.load_gather(dis_v, [sv])
                    ndv = plsc.load_gather(dis_v, [dv])
                    nrm = nsv * ndv
                    inq = (sv >= qlo) & (sv < qlo + QROWS)
                    idx_v[r, sl] = jnp.where(inq, (sv - qlo) * NN + dv, QN + dv)
                    val_v[r, sl] = jnp.where(inq, nrm, 0.0)
                return carry
            lax.fori_loop(0, ROWS, compute, 0)

        prev_dma = None
        for q in range(2):
            panel = 2 * c + q
            qlo = panel * QROWS

            compute_panel(qlo)

            zs = [pltpu.async_copy(
                      zero_v, qbuf.at[pl.ds(s * (QN // 16) + k * 8192, 8192)],
                      sem) for k in range(8)]
            for z in zs:
                z.wait()
            plsc.subcore_barrier()

            scs = [pltpu.async_copy(val_v.at[j], qbuf.at[idx_v.at[j]], sem,
                                    add=True) for j in range(ROWS)]
            for d in scs:
                d.wait()
            plsc.subcore_barrier()

            for chunk in range(4):
                bb = bounce_a if chunk % 2 == 0 else bounce_b
                rws = [pltpu.async_copy(
                           qbuf.at[pl.ds((s * 32 + chunk * 8 + k) * NN, NN)],
                           bb.at[k], sem) for k in range(8)]
                for r in rws:
                    r.wait()
                if prev_dma is not None:
                    prev_dma.wait()
                prev_dma = pltpu.async_copy(
                    bb, at_hbm.at[pl.ds(panel * QROWS + s * 32 + chunk * 8, 8)],
                    sem_w)
        prev_dma.wait()

    return pl.kernel(
        body,
        out_type=jax.ShapeDtypeStruct((NN, NN), jnp.float32),
        mesh=mesh,
        compiler_params=pltpu.CompilerParams(needs_layout_passes=False),
        scratch_types=[
            pltpu.VMEM((EPT,), jnp.int32),
            pltpu.VMEM((EPT,), jnp.int32),
            pltpu.VMEM((ROWS, 128), jnp.int32),
            pltpu.VMEM((ROWS, 128), jnp.float32),
            pltpu.VMEM((NN,), jnp.float32),
            pltpu.VMEM((8192,), jnp.float32),
            pltpu.VMEM((8, NN), jnp.float32),
            pltpu.VMEM((8, NN), jnp.float32),
            pltpu.VMEM_SHARED((QN + NN,), jnp.float32),
            pltpu.VMEM_SHARED((NN,), jnp.float32),
            pltpu.SemaphoreType.DMA,
            pltpu.SemaphoreType.DMA,
        ],
    )(sc_src, sc_dst)


_PH = np.zeros((64, 32), np.float32)
_PH[np.arange(64), np.arange(64) // 2] = 0.5
_GP = np.kron(_PH, _PH)


def _pool_body(rgb_ref, ir_ref, pos_ref, gp_ref, out_ref):
    gp = gp_ref[...]
    x = jnp.concatenate(
        [jnp.dot(rgb_ref[0], gp, preferred_element_type=jnp.float32),
         jnp.dot(ir_ref[0], gp, preferred_element_type=jnp.float32)], axis=1)
    out_ref[0] = x + pos_ref[...]


def _gcn_body(x_ref, wt_ref, bs_ref, at_ref, lnw_ref, lnb_ref, out_ref):
    a = at_ref[...]
    x = x_ref[0]
    for i in range(NL):
        z = jnp.dot(wt_ref[i], x, preferred_element_type=jnp.float32)
        g = jnp.dot(z, a, preferred_element_type=jnp.float32)
        x = jnp.maximum(g + bs_ref[i], 0.0)
    mu = jnp.mean(x, axis=0, keepdims=True)
    xc = x - mu
    var = jnp.mean(xc * xc, axis=0, keepdims=True)
    xn = xc * lax.rsqrt(var + 1e-5)
    out_ref[0] = xn * lnw_ref[...] + lnb_ref[...]


def _resize_body(y_ref, g_ref, rgb_ref, ir_ref):
    g = g_ref[...]
    rgb_ref[0] = jnp.dot(y_ref[0, :, :1024], g,
                         preferred_element_type=jnp.float32)
    ir_ref[0] = jnp.dot(y_ref[0, :, 1024:], g,
                        preferred_element_type=jnp.float32)


def _resize_matrix():
    sf = (np.arange(64) + 0.5) / 2.0 - 0.5
    w = np.maximum(0.0, 1.0 - np.abs(sf[None, :] - np.arange(32)[:, None]))
    w = w / w.sum(axis=0, keepdims=True)
    return w.T.astype(np.float32)


_R = _resize_matrix()
_G = np.kron(_R, _R).T.copy()


def kernel(rgb_fea, ir_fea, edge_index, pos_emb, Ws, Bs, ln_w, ln_b):
    pos_t = jnp.transpose(pos_emb[0])
    wt = jnp.transpose(Ws, (0, 2, 1))
    bs2 = Bs[:, :, None]
    lnw2 = ln_w[:, None]
    lnb2 = ln_b[:, None]

    loop = jnp.arange(NN, dtype=jnp.int32)
    sc_src = jnp.concatenate([edge_index[0], loop])
    sc_dst = jnp.concatenate([edge_index[1], loop])

    at = _sc_build_at(sc_src, sc_dst)

    x0 = pl.pallas_call(
        _pool_body,
        grid=(BSZ,),
        in_specs=[
            pl.BlockSpec((1, DM, 4096), lambda b: (b, 0, 0)),
            pl.BlockSpec((1, DM, 4096), lambda b: (b, 0, 0)),
            pl.BlockSpec((DM, NN), lambda b: (0, 0)),
            pl.BlockSpec((4096, 1024), lambda b: (0, 0)),
        ],
        out_specs=pl.BlockSpec((1, DM, NN), lambda b: (b, 0, 0)),
        out_shape=jax.ShapeDtypeStruct((BSZ, DM, NN), jnp.float32),
    )(rgb_fea.reshape(BSZ, DM, 4096), ir_fea.reshape(BSZ, DM, 4096),
      pos_t, jnp.asarray(_GP))

    y = pl.pallas_call(
        _gcn_body,
        grid=(BSZ,),
        in_specs=[
            pl.BlockSpec((1, DM, NN), lambda b: (b, 0, 0)),
            pl.BlockSpec((NL, DM, DM), lambda b: (0, 0, 0)),
            pl.BlockSpec((NL, DM, 1), lambda b: (0, 0, 0)),
            pl.BlockSpec((NN, NN), lambda b: (0, 0)),
            pl.BlockSpec((DM, 1), lambda b: (0, 0)),
            pl.BlockSpec((DM, 1), lambda b: (0, 0)),
        ],
        out_specs=pl.BlockSpec((1, DM, NN), lambda b: (b, 0, 0)),
        out_shape=jax.ShapeDtypeStruct((BSZ, DM, NN), jnp.float32),
    )(x0, wt, bs2, at, lnw2, lnb2)

    rgb_out, ir_out = pl.pallas_call(
        _resize_body,
        grid=(BSZ,),
        in_specs=[
            pl.BlockSpec((1, DM, NN), lambda b: (b, 0, 0)),
            pl.BlockSpec((1024, 4096), lambda b: (0, 0)),
        ],
        out_specs=[
            pl.BlockSpec((1, DM, 4096), lambda b: (b, 0, 0)),
            pl.BlockSpec((1, DM, 4096), lambda b: (b, 0, 0)),
        ],
        out_shape=[
            jax.ShapeDtypeStruct((BSZ, DM, 4096), jnp.float32),
            jax.ShapeDtypeStruct((BSZ, DM, 4096), jnp.float32),
        ],
    )(y, jnp.asarray(_G))

    return (rgb_out.reshape(BSZ, DM, 64, 64), ir_out.reshape(BSZ, DM, 64, 64))
